# baseline (speedup 1.0000x reference)
import jax
import jax.numpy as jnp
from jax.experimental import pallas as pl
from jax.experimental.pallas import tpu as pltpu

_NFILT = 80
_NBINS = 257


def _body(x_hbm, p_ref, wl1_ref, bl1_ref, wc1_ref, bc1_ref, wc2_ref, bc2_ref,
          wc3_ref, bc3_ref, wa1_ref, ba1_ref, wa2_ref, ba2_ref, g_ref, be_ref,
          wl2_ref, bl2_ref, o_ref, xbuf, fbbuf, sem):
    f32 = jnp.float32
    bf16 = jnp.bfloat16
    B = x_hbm.shape[1]
    T = x_hbm.shape[2]
    G = 4
    nstep = B // G
    j = pl.program_id(0)
    spair = jax.lax.rem(j, 2) * G

    def copy_in(bi, si):
        return pltpu.make_async_copy(x_hbm.at[:, bi, :], xbuf.at[si],
                                     sem.at[si])

    @pl.when(j == 0)
    def _():
        for k in range(G):
            copy_in(k, k).start()

    @pl.when(j + 1 < nstep)
    def _():
        nx = jax.lax.rem(j + 1, 2) * G
        for k in range(G):
            copy_in(G * j + G + k, nx + k).start()

    @pl.when(j == 0)
    def _():
        P = p_ref[...]
        bj, bj1, bj2 = P[:, 0:1], P[:, 1:2], P[:, 2:3]
        ibj, ibj1, ibj2 = P[:, 3:4], P[:, 4:5], P[:, 5:6]
        rowmask = P[:, 6:7]
        I = jax.lax.broadcasted_iota(
            jnp.int32, (_NFILT, _NBINS - 1), 1).astype(f32)
        rise_m = (I >= ibj) & (I < ibj1)
        fall_m = (I >= ibj1) & (I < ibj2)
        d1 = (bj1 - bj) ** 2
        d2 = (bj2 - bj1) ** 2
        rise = (I - bj) / jnp.where(d1 > 0, d1, 1.0)
        fall = (bj2 - I) / jnp.where(d2 > 0, d2, 1.0)
        fbbuf[...] = (jnp.where(rise_m, rise,
                                jnp.where(fall_m, fall, 0.0))
                      * rowmask).astype(bf16)

    def dconv(hv, w_ref, b_ref, d):
        c = hv.shape[0]
        z = jnp.zeros((c, d), f32)
        hm = jnp.concatenate([z, hv[:, :T - d]], axis=1).astype(bf16)
        hp = jnp.concatenate([hv[:, d:], z], axis=1).astype(bf16)
        hcat = jnp.concatenate([hm, hv.astype(bf16), hp], axis=0)
        return (jnp.dot(w_ref[...], hcat, preferred_element_type=f32)
                + b_ref[...])

    def one_batch(xb, k):
        filt = jnp.dot(fbbuf[...], xb[:_NBINS - 1].astype(bf16),
                       preferred_element_type=f32)
        row = jax.lax.broadcasted_iota(jnp.int32, (_NFILT, T), 0)
        filt = jnp.where(row == 0, xb[0:1, :], filt)
        h = jnp.maximum(
            jnp.dot(wl1_ref[...], filt.astype(bf16),
                    preferred_element_type=f32) + bl1_ref[...], 0.0)

        i1 = dconv(h, wc1_ref, bc1_ref, 2)
        i2 = dconv(i1, wc2_ref, bc2_ref, 3) + i1
        i12 = i1 + i2
        i3 = dconv(i12, wc3_ref, bc3_ref, 4) + i12

        a = jnp.tanh(
            jnp.dot(wa1_ref[...], i3.astype(bf16), preferred_element_type=f32)
            + ba1_ref[...])
        e = (jnp.dot(wa2_ref[...], a.astype(bf16), preferred_element_type=f32)
             + ba2_ref[...])
        m = jnp.max(e, axis=1, keepdims=True)
        pexp = jnp.exp(e - m)
        s = jnp.sum(pexp, axis=1, keepdims=True)
        pi3 = pexp * i3
        w1 = jnp.sum(pi3, axis=1, keepdims=True)
        w2 = jnp.sum(pi3 * i3, axis=1, keepdims=True)
        sinv = 1.0 / s
        mean = w1 * sinv
        msq = w2 * sinv
        std = jnp.sqrt(jnp.clip(msq - mean * mean, 1e-9))
        pooled = jnp.concatenate([mean, std], axis=0)

        mu = jnp.mean(pooled, axis=0, keepdims=True)
        var = jnp.mean((pooled - mu) ** 2, axis=0, keepdims=True)
        gn = ((pooled - mu) / jnp.sqrt(var + 1e-5) * g_ref[...]
              + be_ref[...])
        out = jax.lax.dot_general(gn.astype(bf16), wl2_ref[...],
                                  (((0,), (0,)), ((), ())),
                                  preferred_element_type=f32)
        o_ref[0, k] = (out + bl2_ref[...])[0]

    for k in range(G):
        copy_in(G * j + k, spair + k).wait()
        one_batch(xbuf[spair + k], k)


def kernel(x, binpoints, w_lin1, b_lin1, w_conv1, b_conv1, w_conv2, b_conv2,
           w_conv3, b_conv3, w_asp1, b_asp1, w_asp2, b_asp2, gamma, beta,
           w_lin2, b_lin2):
    f32 = jnp.float32
    bf16 = jnp.bfloat16
    B, T, F = x.shape
    xt = jnp.transpose(x, (2, 0, 1))

    bp = binpoints.astype(f32)
    ib = jnp.floor(bp)
    P = jnp.stack([
        bp[:_NFILT], bp[1:_NFILT + 1], bp[2:_NFILT + 2],
        ib[:_NFILT], ib[1:_NFILT + 1], ib[2:_NFILT + 2],
        (jnp.arange(_NFILT) < _NFILT - 1).astype(f32),
        jnp.zeros((_NFILT,), f32),
    ], axis=1)

    wl1 = w_lin1.astype(bf16)
    stack_taps = lambda w: jnp.transpose(w, (0, 2, 1)).reshape(
        w.shape[0], 3 * w.shape[1]).astype(bf16)
    wc1 = stack_taps(w_conv1)
    wc2 = stack_taps(w_conv2)
    wc3 = stack_taps(w_conv3)
    wa1 = w_asp1.astype(bf16)
    wa2 = w_asp2.astype(bf16)
    wl2t = w_lin2.T.astype(bf16)

    col = lambda v: v.reshape(-1, 1)
    row = lambda v: v.reshape(1, -1)
    full = lambda arr: pl.BlockSpec(arr.shape, lambda j: (0,) * arr.ndim)
    operands = [P, wl1, col(b_lin1), wc1, col(b_conv1), wc2, col(b_conv2),
                wc3, col(b_conv3), wa1, col(b_asp1), wa2, col(b_asp2),
                col(gamma), col(beta), wl2t, row(b_lin2)]

    G = 4
    out = pl.pallas_call(
        _body,
        grid=(B // G,),
        in_specs=[pl.BlockSpec(memory_space=pl.ANY)]
                 + [full(a) for a in operands],
        out_specs=pl.BlockSpec((1, G, 512), lambda j: (j, 0, 0)),
        out_shape=jax.ShapeDtypeStruct((B // G, G, 512), f32),
        scratch_shapes=[
            pltpu.VMEM((2 * G, F, T), f32),
            pltpu.VMEM((_NFILT, _NBINS - 1), bf16),
            pltpu.SemaphoreType.DMA((2 * G,)),
        ],
        compiler_params=pltpu.CompilerParams(
            dimension_semantics=("arbitrary",),
            vmem_limit_bytes=55 * 1024 * 1024,
        ),
    )(xt, *operands)
    return out.reshape(B, 512)


# speedup vs baseline: 1.0413x; 1.0413x over previous
import jax
import jax.numpy as jnp
from jax.experimental import pallas as pl
from jax.experimental.pallas import tpu as pltpu

_NFILT = 80
_NBINS = 257


def _body(x_hbm, p_ref, wl1_ref, bl1_ref, wc1_ref, bc1_ref, wc2_ref, bc2_ref,
          wc3_ref, bc3_ref, wa1_ref, ba1_ref, wa2_ref, ba2_ref, g_ref, be_ref,
          wl2_ref, bl2_ref, o_ref, xbuf, fbbuf, sem):
    f32 = jnp.float32
    bf16 = jnp.bfloat16
    B = x_hbm.shape[1]
    T = x_hbm.shape[2]
    G = 2
    nstep = B // G
    j = pl.program_id(0)
    spair = jax.lax.rem(j, 2) * G

    def copy_in(bi, si):
        return pltpu.make_async_copy(x_hbm.at[:, bi, :], xbuf.at[si],
                                     sem.at[si])

    @pl.when(j == 0)
    def _():
        for k in range(G):
            copy_in(k, k).start()

    @pl.when(j + 1 < nstep)
    def _():
        nx = jax.lax.rem(j + 1, 2) * G
        for k in range(G):
            copy_in(G * j + G + k, nx + k).start()

    @pl.when(j == 0)
    def _():
        P = p_ref[...]
        bj, bj1, bj2 = P[:, 0:1], P[:, 1:2], P[:, 2:3]
        ibj, ibj1, ibj2 = P[:, 3:4], P[:, 4:5], P[:, 5:6]
        rowmask = P[:, 6:7]
        I = jax.lax.broadcasted_iota(
            jnp.int32, (_NFILT, _NBINS - 1), 1).astype(f32)
        rise_m = (I >= ibj) & (I < ibj1)
        fall_m = (I >= ibj1) & (I < ibj2)
        d1 = (bj1 - bj) ** 2
        d2 = (bj2 - bj1) ** 2
        rise = (I - bj) / jnp.where(d1 > 0, d1, 1.0)
        fall = (bj2 - I) / jnp.where(d2 > 0, d2, 1.0)
        fbbuf[...] = (jnp.where(rise_m, rise,
                                jnp.where(fall_m, fall, 0.0))
                      * rowmask).astype(bf16)

    def dconv(hv, w_ref, b_ref, d):
        c = hv.shape[0]
        z = jnp.zeros((c, d), f32)
        hm = jnp.concatenate([z, hv[:, :T - d]], axis=1).astype(bf16)
        hp = jnp.concatenate([hv[:, d:], z], axis=1).astype(bf16)
        hcat = jnp.concatenate([hm, hv.astype(bf16), hp], axis=0)
        return (jnp.dot(w_ref[...], hcat, preferred_element_type=f32)
                + b_ref[...])

    def one_batch(xb, k):
        filt = jnp.dot(fbbuf[...], xb[:_NBINS - 1].astype(bf16),
                       preferred_element_type=f32)
        row = jax.lax.broadcasted_iota(jnp.int32, (_NFILT, T), 0)
        filt = jnp.where(row == 0, xb[0:1, :], filt)
        h = jnp.maximum(
            jnp.dot(wl1_ref[...], filt.astype(bf16),
                    preferred_element_type=f32) + bl1_ref[...], 0.0)

        i1 = dconv(h, wc1_ref, bc1_ref, 2)
        i2 = dconv(i1, wc2_ref, bc2_ref, 3) + i1
        i12 = i1 + i2
        i3 = dconv(i12, wc3_ref, bc3_ref, 4) + i12

        a = jnp.tanh(
            jnp.dot(wa1_ref[...], i3.astype(bf16), preferred_element_type=f32)
            + ba1_ref[...])
        e = (jnp.dot(wa2_ref[...], a.astype(bf16), preferred_element_type=f32)
             + ba2_ref[...])
        m = jnp.max(e, axis=1, keepdims=True)
        pexp = jnp.exp(e - m)
        s = jnp.sum(pexp, axis=1, keepdims=True)
        pi3 = pexp * i3
        w1 = jnp.sum(pi3, axis=1, keepdims=True)
        w2 = jnp.sum(pi3 * i3, axis=1, keepdims=True)
        sinv = 1.0 / s
        mean = w1 * sinv
        msq = w2 * sinv
        std = jnp.sqrt(jnp.clip(msq - mean * mean, 1e-9))
        pooled = jnp.concatenate([mean, std], axis=0)

        mu = jnp.mean(pooled, axis=0, keepdims=True)
        var = jnp.mean((pooled - mu) ** 2, axis=0, keepdims=True)
        gn = ((pooled - mu) / jnp.sqrt(var + 1e-5) * g_ref[...]
              + be_ref[...])
        out = jax.lax.dot_general(gn.astype(bf16), wl2_ref[...],
                                  (((0,), (0,)), ((), ())),
                                  preferred_element_type=f32)
        o_ref[0, k] = (out + bl2_ref[...])[0]

    for k in range(G):
        copy_in(G * j + k, spair + k).wait()
        one_batch(xbuf[spair + k], k)


def kernel(x, binpoints, w_lin1, b_lin1, w_conv1, b_conv1, w_conv2, b_conv2,
           w_conv3, b_conv3, w_asp1, b_asp1, w_asp2, b_asp2, gamma, beta,
           w_lin2, b_lin2):
    f32 = jnp.float32
    bf16 = jnp.bfloat16
    B, T, F = x.shape
    xt = jnp.transpose(x, (2, 0, 1))

    bp = binpoints.astype(f32)
    ib = jnp.floor(bp)
    P = jnp.stack([
        bp[:_NFILT], bp[1:_NFILT + 1], bp[2:_NFILT + 2],
        ib[:_NFILT], ib[1:_NFILT + 1], ib[2:_NFILT + 2],
        (jnp.arange(_NFILT) < _NFILT - 1).astype(f32),
        jnp.zeros((_NFILT,), f32),
    ], axis=1)

    wl1 = w_lin1.astype(bf16)
    stack_taps = lambda w: jnp.transpose(w, (0, 2, 1)).reshape(
        w.shape[0], 3 * w.shape[1]).astype(bf16)
    wc1 = stack_taps(w_conv1)
    wc2 = stack_taps(w_conv2)
    wc3 = stack_taps(w_conv3)
    wa1 = w_asp1.astype(bf16)
    wa2 = w_asp2.astype(bf16)
    wl2t = w_lin2.T.astype(bf16)

    col = lambda v: v.reshape(-1, 1)
    row = lambda v: v.reshape(1, -1)
    full = lambda arr: pl.BlockSpec(arr.shape, lambda j: (0,) * arr.ndim)
    operands = [P, wl1, col(b_lin1), wc1, col(b_conv1), wc2, col(b_conv2),
                wc3, col(b_conv3), wa1, col(b_asp1), wa2, col(b_asp2),
                col(gamma), col(beta), wl2t, row(b_lin2)]

    G = 2
    out = pl.pallas_call(
        _body,
        grid=(B // G,),
        in_specs=[pl.BlockSpec(memory_space=pl.ANY)]
                 + [full(a) for a in operands],
        out_specs=pl.BlockSpec((1, G, 512), lambda j: (j, 0, 0)),
        out_shape=jax.ShapeDtypeStruct((B // G, G, 512), f32),
        scratch_shapes=[
            pltpu.VMEM((2 * G, F, T), f32),
            pltpu.VMEM((_NFILT, _NBINS - 1), bf16),
            pltpu.SemaphoreType.DMA((2 * G,)),
        ],
        compiler_params=pltpu.CompilerParams(
            dimension_semantics=("arbitrary",),
            vmem_limit_bytes=48 * 1024 * 1024,
        ),
    )(xt, *operands)
    return out.reshape(B, 512)


# speedup vs baseline: 1.0425x; 1.0011x over previous
import jax
import jax.numpy as jnp
from jax.experimental import pallas as pl
from jax.experimental.pallas import tpu as pltpu

_NFILT = 80
_NBINS = 257


def _body(x_hbm, p_ref, wl1_ref, bl1_ref, wc1_ref, bc1_ref, wc2_ref, bc2_ref,
          wc3_ref, bc3_ref, wa1_ref, ba1_ref, wa2_ref, ba2_ref, g_ref, be_ref,
          wl2_ref, bl2_ref, o_ref, xbuf, fbbuf, sem):
    f32 = jnp.float32
    bf16 = jnp.bfloat16
    B = x_hbm.shape[1]
    T = x_hbm.shape[2]
    G = 2
    nstep = B // G
    j = pl.program_id(0)
    spair = jax.lax.rem(j, 2) * G

    def copy_in(bi, si):
        return pltpu.make_async_copy(x_hbm.at[:, bi, :], xbuf.at[si],
                                     sem.at[si])

    @pl.when(j == 0)
    def _():
        for k in range(G):
            copy_in(k, k).start()

    @pl.when(j + 1 < nstep)
    def _():
        nx = jax.lax.rem(j + 1, 2) * G
        for k in range(G):
            copy_in(G * j + G + k, nx + k).start()

    @pl.when(j == 0)
    def _():
        P = p_ref[...]
        bj, bj1, bj2 = P[:, 0:1], P[:, 1:2], P[:, 2:3]
        ibj, ibj1, ibj2 = P[:, 3:4], P[:, 4:5], P[:, 5:6]
        rowmask = P[:, 6:7]
        I = jax.lax.broadcasted_iota(
            jnp.int32, (_NFILT, _NBINS - 1), 1).astype(f32)
        rise_m = (I >= ibj) & (I < ibj1)
        fall_m = (I >= ibj1) & (I < ibj2)
        d1 = (bj1 - bj) ** 2
        d2 = (bj2 - bj1) ** 2
        rise = (I - bj) / jnp.where(d1 > 0, d1, 1.0)
        fall = (bj2 - I) / jnp.where(d2 > 0, d2, 1.0)
        fbbuf[...] = (jnp.where(rise_m, rise,
                                jnp.where(fall_m, fall, 0.0))
                      * rowmask).astype(bf16)

    def dconv(hv, w_ref, b_ref, d):
        c = hv.shape[0]
        z = jnp.zeros((c, d), f32)
        hm = jnp.concatenate([z, hv[:, :T - d]], axis=1).astype(bf16)
        hp = jnp.concatenate([hv[:, d:], z], axis=1).astype(bf16)
        hcat = jnp.concatenate([hm, hv.astype(bf16), hp], axis=0)
        return (jnp.dot(w_ref[...], hcat, preferred_element_type=f32)
                + b_ref[...])

    def one_batch(xb, k):
        filt = jnp.dot(fbbuf[...], xb[:_NBINS - 1].astype(bf16),
                       preferred_element_type=f32)
        row = jax.lax.broadcasted_iota(jnp.int32, (_NFILT, T), 0)
        filt = jnp.where(row == 0, xb[0:1, :], filt)
        h = jnp.maximum(
            jnp.dot(wl1_ref[...], filt.astype(bf16),
                    preferred_element_type=f32) + bl1_ref[...], 0.0)

        i1 = dconv(h, wc1_ref, bc1_ref, 2)
        i2 = dconv(i1, wc2_ref, bc2_ref, 3) + i1
        i12 = i1 + i2
        i3 = dconv(i12, wc3_ref, bc3_ref, 4) + i12

        a = jnp.tanh(
            jnp.dot(wa1_ref[...], i3.astype(bf16), preferred_element_type=f32)
            + ba1_ref[...])
        e = (jnp.dot(wa2_ref[...], a.astype(bf16), preferred_element_type=f32)
             + ba2_ref[...])
        m = jnp.max(e, axis=1, keepdims=True)
        pexp = jnp.exp(e - m)
        s = jnp.sum(pexp, axis=1, keepdims=True)
        pi3 = pexp * i3
        w1 = jnp.sum(pi3, axis=1, keepdims=True)
        w2 = jnp.sum(pi3 * i3, axis=1, keepdims=True)
        sinv = 1.0 / s
        mean = w1 * sinv
        msq = w2 * sinv
        std = jnp.sqrt(jnp.clip(msq - mean * mean, 1e-9))
        pooled = jnp.concatenate([mean, std], axis=0)

        mu = jnp.mean(pooled, axis=0, keepdims=True)
        var = jnp.mean((pooled - mu) ** 2, axis=0, keepdims=True)
        gn = ((pooled - mu) / jnp.sqrt(var + 1e-5) * g_ref[...]
              + be_ref[...])
        out = jax.lax.dot_general(gn.astype(bf16), wl2_ref[...],
                                  (((0,), (0,)), ((), ())),
                                  preferred_element_type=f32)
        o_ref[0, k] = (out + bl2_ref[...])[0]

    for k in range(G):
        copy_in(G * j + k, spair + k).wait()
        one_batch(xbuf[spair + k], k)


def kernel(x, binpoints, w_lin1, b_lin1, w_conv1, b_conv1, w_conv2, b_conv2,
           w_conv3, b_conv3, w_asp1, b_asp1, w_asp2, b_asp2, gamma, beta,
           w_lin2, b_lin2):
    f32 = jnp.float32
    bf16 = jnp.bfloat16
    B, T, F = x.shape
    xt = jnp.transpose(x, (2, 0, 1))

    bp = binpoints.astype(f32)
    ib = jnp.floor(bp)
    P = jnp.stack([
        bp[:_NFILT], bp[1:_NFILT + 1], bp[2:_NFILT + 2],
        ib[:_NFILT], ib[1:_NFILT + 1], ib[2:_NFILT + 2],
        (jnp.arange(_NFILT) < _NFILT - 1).astype(f32),
        jnp.zeros((_NFILT,), f32),
    ], axis=1)

    wl1 = w_lin1.astype(bf16)
    stack_taps = lambda w: jnp.transpose(w, (0, 2, 1)).reshape(
        w.shape[0], 3 * w.shape[1]).astype(bf16)
    wc1 = stack_taps(w_conv1)
    wc2 = stack_taps(w_conv2)
    wc3 = stack_taps(w_conv3)
    wa1 = w_asp1.astype(bf16)
    wa2 = w_asp2.astype(bf16)
    wl2t = w_lin2.T.astype(bf16)

    col = lambda v: v.reshape(-1, 1)
    row = lambda v: v.reshape(1, -1)
    full = lambda arr: pl.BlockSpec(arr.shape, lambda j: (0,) * arr.ndim)
    operands = [P, wl1, col(b_lin1), wc1, col(b_conv1), wc2, col(b_conv2),
                wc3, col(b_conv3), wa1, col(b_asp1), wa2, col(b_asp2),
                col(gamma), col(beta), wl2t, row(b_lin2)]

    G = 2
    out = pl.pallas_call(
        _body,
        grid=(B // G,),
        in_specs=[pl.BlockSpec(memory_space=pl.ANY)]
                 + [full(a) for a in operands],
        out_specs=pl.BlockSpec((1, G, 512), lambda j: (j, 0, 0)),
        out_shape=jax.ShapeDtypeStruct((B // G, G, 512), f32),
        scratch_shapes=[
            pltpu.VMEM((2 * G, F, T), f32),
            pltpu.VMEM((_NFILT, _NBINS - 1), bf16),
            pltpu.SemaphoreType.DMA((2 * G,)),
        ],
        compiler_params=pltpu.CompilerParams(
            dimension_semantics=("arbitrary",),
            vmem_limit_bytes=48 * 1024 * 1024,
            flags={"XLA_TPU_STORE_TO_LOAD_FORWARDING_WINDOW": 12288},
        ),
    )(xt, *operands)
    return out.reshape(B, 512)
